# baseline (speedup 1.0000x reference)
import functools

import jax
import jax.numpy as jnp
from jax import lax
from jax.experimental import pallas as pl
from jax.experimental.pallas import tpu as pltpu
from jax.experimental.pallas import tpu_sc as plsc

N = 10000
D = 128
K = 8
NP = 10240
NC, NS, L = 2, 16, 16
NW = NC * NS
CHUNK = 128
CPT = 80
EP = NW * CPT * CHUNK
RPS = NP // NS
GR = 128
SHARE = EP // NW
DT = SHARE // GR
NRING = 4
BLK = 256


def _mesh():
    return plsc.VectorSubcoreMesh(core_axis_name="c", subcore_axis_name="s")


def _sc_hist(rows2d, colp2d):
    @functools.partial(
        pl.kernel,
        out_type=jax.ShapeDtypeStruct((NW, NP), jnp.float32),
        mesh=_mesh(),
        compiler_params=pltpu.CompilerParams(needs_layout_passes=False),
        scratch_types=[
            pltpu.VMEM((CPT, CHUNK), jnp.int32),
            pltpu.VMEM((CPT, CHUNK), jnp.int32),
            pltpu.VMEM((NP,), jnp.float32),
        ],
    )
    def hist_kernel(rows_hbm, colp_hbm, h_hbm, ridx, cidx, hist):
        cid = lax.axis_index("c")
        sid = lax.axis_index("s")
        wid = cid * NS + sid
        pltpu.sync_copy(rows_hbm.at[pl.ds(wid * CPT, CPT)], ridx)
        pltpu.sync_copy(colp_hbm.at[pl.ds(wid * CPT, CPT)], cidx)
        zeros16 = jnp.zeros((L,), jnp.float32)

        def zbody(i, carry):
            hist[pl.ds(i * L, L)] = zeros16
            return carry

        lax.fori_loop(0, NP // L, zbody, 0)
        ones16 = jnp.ones((L,), jnp.float32)
        nvec = jnp.full((L,), N, jnp.int32)

        def cbody(c, carry):
            for i in range(CHUNK // L):
                rv = ridx[c, pl.ds(i * L, L)]
                cv = cidx[c, pl.ds(i * L, L)]
                rv = jnp.where(cv < nvec, rv, nvec)
                plsc.addupdate_scatter(hist, [rv], ones16)
            return carry

        lax.fori_loop(0, CPT, cbody, 0)
        pltpu.sync_copy(hist, h_hbm.at[wid])

    return hist_kernel(rows2d, colp2d)


def _sc_edge(z2d, rows1d, colp1d):
    @functools.partial(
        pl.kernel,
        out_type=jax.ShapeDtypeStruct((NC, NP, D), jnp.float32),
        mesh=_mesh(),
        compiler_params=pltpu.CompilerParams(needs_layout_passes=False),
        scratch_types=[
            pltpu.VMEM((DT, GR), jnp.int32),
            pltpu.VMEM((DT, GR), jnp.int32),
            pltpu.VMEM((GR, D), jnp.float32),
            pltpu.VMEM_SHARED((NP, D), jnp.float32),
            pltpu.SemaphoreType.DMA,
        ],
    )
    def edge_kernel(
        z_hbm, rows_hbm, colp_hbm, p_hbm, ridx, cidx, buf, acc, gsem
    ):
        cid = lax.axis_index("c")
        sid = lax.axis_index("s")
        wid = cid * NS + sid

        pltpu.sync_copy(rows_hbm.at[pl.ds(wid * DT, DT)], ridx)
        pltpu.sync_copy(colp_hbm.at[pl.ds(wid * DT, DT)], cidx)

        zeros16 = jnp.zeros((L,), jnp.float32)

        def zbody(j, carry):
            for i in range(D // L):
                buf[j, pl.ds(i * L, L)] = zeros16
            return carry

        lax.fori_loop(0, GR, zbody, 0)
        for t in range(RPS // GR):
            pltpu.sync_copy(buf, acc.at[pl.ds(sid * RPS + t * GR, GR)])
        plsc.subcore_barrier()

        def dbody(d, carry):
            pltpu.async_copy(z_hbm.at[ridx.at[d]], buf, gsem).wait()
            pltpu.sync_copy(buf, acc.at[cidx.at[d]], add=True)
            return carry

        lax.fori_loop(0, DT, dbody, 0)
        plsc.subcore_barrier()
        for t in range(RPS // GR):
            off = sid * RPS + t * GR
            pltpu.sync_copy(acc.at[pl.ds(off, GR)], buf)
            pltpu.sync_copy(buf, p_hbm.at[cid, pl.ds(off, GR)])

    return edge_kernel(z2d, rows1d, colp1d)


def _tc_init(h, x_pad):
    def body(h_ref, x_ref, z_ref, dis_ref):
        counts = jnp.sum(h_ref[...], axis=0)
        dis = lax.rsqrt(counts + 1.0)[:, None]
        z_ref[...] = dis * x_ref[...]
        dis_ref[...] = dis

    return pl.pallas_call(
        body,
        grid=(NP // BLK,),
        in_specs=[
            pl.BlockSpec((NW, BLK), lambda i: (0, i)),
            pl.BlockSpec((BLK, D), lambda i: (i, 0)),
        ],
        out_specs=[
            pl.BlockSpec((BLK, D), lambda i: (i, 0)),
            pl.BlockSpec((BLK, 1), lambda i: (i, 0)),
        ],
        out_shape=[
            jax.ShapeDtypeStruct((NP, D), jnp.float32),
            jax.ShapeDtypeStruct((NP, 1), jnp.float32),
        ],
    )(h, x_pad)


def _tc_combine(p, z, dis):
    def body(p_ref, z_ref, dis_ref, out_ref, z2_ref):
        s = p_ref[0] + p_ref[1] + z_ref[...]
        dd = dis_ref[...]
        o = dd * s
        out_ref[...] = o
        z2_ref[...] = dd * o

    return pl.pallas_call(
        body,
        grid=(NP // BLK,),
        in_specs=[
            pl.BlockSpec((NC, BLK, D), lambda i: (0, i, 0)),
            pl.BlockSpec((BLK, D), lambda i: (i, 0)),
            pl.BlockSpec((BLK, 1), lambda i: (i, 0)),
        ],
        out_specs=[
            pl.BlockSpec((BLK, D), lambda i: (i, 0)),
            pl.BlockSpec((BLK, D), lambda i: (i, 0)),
        ],
        out_shape=[
            jax.ShapeDtypeStruct((NP, D), jnp.float32),
            jax.ShapeDtypeStruct((NP, D), jnp.float32),
        ],
    )(p, z, dis)


def kernel(feature, edge_index):
    row = edge_index[0]
    col = edge_index[1]
    colp = jnp.where(row == col, N, col).astype(jnp.int32)
    pad = EP - row.shape[0]
    rows_p = jnp.concatenate([row.astype(jnp.int32), jnp.full((pad,), N, jnp.int32)])
    colp_p = jnp.concatenate([colp, jnp.full((pad,), N, jnp.int32)])
    rows2d = rows_p.reshape(EP // CHUNK, CHUNK)
    colp2d = colp_p.reshape(EP // CHUNK, CHUNK)
    rowsg = rows_p.reshape(EP // GR, GR)
    colpg = colp_p.reshape(EP // GR, GR)
    x_pad = jnp.pad(feature, ((0, NP - N), (0, 0)))

    h = _sc_hist(rows2d, colp2d)
    z, dis = _tc_init(h, x_pad)
    outs = [feature]
    for _ in range(K):
        p = _sc_edge(z, rowsg, colpg)
        o, z = _tc_combine(p, z, dis)
        outs.append(o[:N])
    return jnp.concatenate(outs, axis=1)
